# Initial kernel scaffold; baseline (speedup 1.0000x reference)
import functools

import jax
import jax.numpy as jnp
from jax import lax
from jax.experimental import pallas as pl
from jax.experimental.pallas import tpu as pltpu
from jax.experimental.pallas import tpu_sc as plsc

N = 10000
E = 320000
D = 128

NC = 2
NS = 16
NW = NC * NS
EPW = E // NW
K = 80
NCHUNK = EPW // K
RPT = N // NS
CPY = 125
NCPY = RPT // CPY


def _sc_agg_body(h_hbm, src_hbm, dst_hbm, out_hbm,
                 acc_sh, src_v, dst_v, rows_v, buf_v, sem):
    c = lax.axis_index("c")
    s = lax.axis_index("s")
    wid = c * NS + s

    @pl.when(c == 0)
    def _():
        for j in range(NCPY):
            r0 = s * RPT + j * CPY
            pltpu.sync_copy(h_hbm.at[pl.ds(r0, CPY), :], buf_v)
            pltpu.sync_copy(buf_v, acc_sh.at[pl.ds(r0, CPY), :])

    @pl.when(c != 0)
    def _():
        def zrow(r, carry):
            for cc in range(D // 16):
                buf_v[r, pl.ds(cc * 16, 16)] = jnp.zeros((16,), jnp.float32)
            return carry
        lax.fori_loop(0, CPY, zrow, 0)
        for j in range(NCPY):
            r0 = s * RPT + j * CPY
            pltpu.sync_copy(buf_v, acc_sh.at[pl.ds(r0, CPY), :])

    plsc.subcore_barrier()

    base = wid * EPW

    def chunk(j, carry):
        off = pl.multiple_of(base + j * K, 8)
        pltpu.sync_copy(src_hbm.at[pl.ds(off, K)], src_v)
        pltpu.sync_copy(dst_hbm.at[pl.ds(off, K)], dst_v)
        pltpu.async_copy(h_hbm.at[src_v], rows_v, sem).wait()
        pltpu.sync_copy(rows_v, acc_sh.at[dst_v], add=True)
        return carry

    lax.fori_loop(0, NCHUNK, chunk, 0)

    plsc.subcore_barrier()

    for j in range(NCPY):
        r0 = s * RPT + j * CPY
        pltpu.sync_copy(acc_sh.at[pl.ds(r0, CPY), :], buf_v)
        pltpu.sync_copy(buf_v, out_hbm.at[c, pl.ds(r0, CPY), :])


_sc_agg = pl.kernel(
    _sc_agg_body,
    out_type=jax.ShapeDtypeStruct((NC, N, D), jnp.float32),
    mesh=plsc.VectorSubcoreMesh(core_axis_name="c", subcore_axis_name="s",
                                num_cores=NC, num_subcores=NS),
    scratch_types=[
        pltpu.VMEM_SHARED((N, D), jnp.float32),
        pltpu.VMEM((K,), jnp.int32),
        pltpu.VMEM((K,), jnp.int32),
        pltpu.VMEM((K, D), jnp.float32),
        pltpu.VMEM((CPY, D), jnp.float32),
        pltpu.SemaphoreType.DMA,
    ],
)

BN = 1000


def _mlp_body(relu_out, p_ref, w1_ref, b1_ref, w2_ref, b2_ref, o_ref):
    z = p_ref[0] + p_ref[1]
    t = jnp.maximum(
        jnp.dot(z, w1_ref[...], preferred_element_type=jnp.float32)
        + b1_ref[...], 0.0)
    o = jnp.dot(t, w2_ref[...], preferred_element_type=jnp.float32) + b2_ref[...]
    if relu_out:
        o = jnp.maximum(o, 0.0)
    o_ref[...] = o


def _mlp(p, w1, b1, w2, b2, relu_out):
    return pl.pallas_call(
        functools.partial(_mlp_body, relu_out),
        grid=(N // BN,),
        in_specs=[
            pl.BlockSpec((NC, BN, D), lambda i: (0, i, 0)),
            pl.BlockSpec((D, D), lambda i: (0, 0)),
            pl.BlockSpec((1, D), lambda i: (0, 0)),
            pl.BlockSpec((D, D), lambda i: (0, 0)),
            pl.BlockSpec((1, D), lambda i: (0, 0)),
        ],
        out_specs=pl.BlockSpec((BN, D), lambda i: (i, 0)),
        out_shape=jax.ShapeDtypeStruct((N, D), jnp.float32),
    )(p, w1, b1.reshape(1, D), w2, b2.reshape(1, D))


def kernel(x, edge_index, W1_0, b1_0, W2_0, b2_0, W1_1, b1_1, W2_1, b2_1):
    src = edge_index[0]
    dst = edge_index[1]
    p = _sc_agg(x, src, dst)
    h = _mlp(p, W1_0, b1_0, W2_0, b2_0, relu_out=True)
    q = _sc_agg(h, src, dst)
    out = _mlp(q, W1_1, b1_1, W2_1, b2_1, relu_out=False)
    return out


# speedup vs baseline: 5.0567x; 5.0567x over previous
import functools

import jax
import jax.numpy as jnp
from jax import lax
from jax.experimental import pallas as pl
from jax.experimental.pallas import tpu as pltpu
from jax.experimental.pallas import tpu_sc as plsc

N = 10000
NP = 10240
E = 320000
D = 128

NC = 2
NS = 16
NW = NC * NS
EPW = E // NW
K = 80
NCHUNK = EPW // K
RPT = NP // NS
CPY = 128
NCPY = RPT // CPY


def _sc_agg_body(h_hbm, src_hbm, dst_hbm, out_hbm,
                 acc_sh, src_v, dst_v, rows_v, buf_v, sem):
    c = lax.axis_index("c")
    s = lax.axis_index("s")
    wid = c * NS + s

    @pl.when(c == 0)
    def _():
        for j in range(NCPY):
            r0 = s * RPT + j * CPY
            pltpu.sync_copy(h_hbm.at[pl.ds(r0, CPY), :], buf_v)
            pltpu.sync_copy(buf_v, acc_sh.at[pl.ds(r0, CPY), :])

    @pl.when(c != 0)
    def _():
        def zrow(r, carry):
            for cc in range(D // 16):
                buf_v[r, pl.ds(cc * 16, 16)] = jnp.zeros((16,), jnp.float32)
            return carry
        lax.fori_loop(0, CPY, zrow, 0)
        for j in range(NCPY):
            r0 = s * RPT + j * CPY
            pltpu.sync_copy(buf_v, acc_sh.at[pl.ds(r0, CPY), :])

    plsc.subcore_barrier()

    base = wid * EPW

    def chunk(j, carry):
        off = pl.multiple_of(base + j * K, 8)
        pltpu.sync_copy(src_hbm.at[pl.ds(off, K)], src_v)
        pltpu.sync_copy(dst_hbm.at[pl.ds(off, K)], dst_v)
        pltpu.async_copy(h_hbm.at[src_v], rows_v, sem).wait()
        pltpu.sync_copy(rows_v, acc_sh.at[dst_v], add=True)
        return carry

    lax.fori_loop(0, NCHUNK, chunk, 0)

    plsc.subcore_barrier()

    for j in range(NCPY):
        r0 = s * RPT + j * CPY
        pltpu.sync_copy(acc_sh.at[pl.ds(r0, CPY), :], buf_v)
        pltpu.sync_copy(buf_v, out_hbm.at[c, pl.ds(r0, CPY), :])


_sc_agg = pl.kernel(
    _sc_agg_body,
    out_type=jax.ShapeDtypeStruct((NC, NP, D), jnp.float32),
    mesh=plsc.VectorSubcoreMesh(core_axis_name="c", subcore_axis_name="s",
                                num_cores=NC, num_subcores=NS),
    scratch_types=[
        pltpu.VMEM_SHARED((NP, D), jnp.float32),
        pltpu.VMEM((K,), jnp.int32),
        pltpu.VMEM((K,), jnp.int32),
        pltpu.VMEM((K, D), jnp.float32),
        pltpu.VMEM((CPY, D), jnp.float32),
        pltpu.SemaphoreType.DMA,
    ],
)

BN = 1024


def _mlp_body(relu_out, p_ref, w1_ref, b1_ref, w2_ref, b2_ref, o_ref):
    z = p_ref[0] + p_ref[1]
    t = jnp.maximum(
        jnp.dot(z, w1_ref[...], preferred_element_type=jnp.float32)
        + b1_ref[...], 0.0)
    o = jnp.dot(t, w2_ref[...], preferred_element_type=jnp.float32) + b2_ref[...]
    if relu_out:
        o = jnp.maximum(o, 0.0)
    o_ref[...] = o


def _mlp(p, w1, b1, w2, b2, relu_out):
    return pl.pallas_call(
        functools.partial(_mlp_body, relu_out),
        grid=(NP // BN,),
        in_specs=[
            pl.BlockSpec((NC, BN, D), lambda i: (0, i, 0)),
            pl.BlockSpec((D, D), lambda i: (0, 0)),
            pl.BlockSpec((1, D), lambda i: (0, 0)),
            pl.BlockSpec((D, D), lambda i: (0, 0)),
            pl.BlockSpec((1, D), lambda i: (0, 0)),
        ],
        out_specs=pl.BlockSpec((BN, D), lambda i: (i, 0)),
        out_shape=jax.ShapeDtypeStruct((NP, D), jnp.float32),
    )(p, w1, b1.reshape(1, D), w2, b2.reshape(1, D))


def kernel(x, edge_index, W1_0, b1_0, W2_0, b2_0, W1_1, b1_1, W2_1, b2_1):
    src = edge_index[0]
    dst = edge_index[1]
    xp = jnp.pad(x, ((0, NP - N), (0, 0)))
    p = _sc_agg(xp, src, dst)
    h = _mlp(p, W1_0, b1_0, W2_0, b2_0, relu_out=True)
    q = _sc_agg(h, src, dst)
    out = _mlp(q, W1_1, b1_1, W2_1, b2_1, relu_out=False)
    return out[:N]
